# Initial kernel scaffold; baseline (speedup 1.0000x reference)
import jax
import jax.numpy as jnp
from jax.experimental import pallas as pl


def kernel(batch_features, embedding):
    raise NotImplementedError("write your pallas kernel here")


# speedup vs baseline: 2.8198x; 2.8198x over previous
import jax
import jax.numpy as jnp
from jax import lax
from jax.experimental import pallas as pl
from jax.experimental.pallas import tpu as pltpu
from jax.experimental.pallas import tpu_sc as plsc

NUM_CORES = 2
NUM_SUBCORES = 16
NUM_WORKERS = NUM_CORES * NUM_SUBCORES
LANES = 16

EMB_DIM = 32
FEAT = 4
OUT_DIM = EMB_DIM + FEAT - 1

B_FULL = 1024
B_BLOCK = 128
N_BBLOCKS = B_FULL // B_BLOCK
SEQ = 50
WORDS = 20
N_CHUNKS = SEQ * N_BBLOCKS
CHUNKS_PER_WORKER = -(-N_CHUNKS // NUM_WORKERS)

GROUPS = B_BLOCK // LANES


def _sc_body(feat_hbm, table_hbm, out_hbm, featp_v, idx_v, grows_v, out_v, sem):
    wid = lax.axis_index("s") * NUM_CORES + lax.axis_index("c")

    def chunk_body(k, carry):
        c = wid + k * NUM_WORKERS

        @pl.when(c < N_CHUNKS)
        def _():
            s = c // N_BBLOCKS
            b0 = (c % N_BBLOCKS) * B_BLOCK

            def jp_body(jp, carry2):
                pltpu.sync_copy(
                    feat_hbm.at[s, pl.ds(jp * 8, 8), pl.ds(b0, B_BLOCK)],
                    featp_v,
                )
                for jj in range(2):
                    j = jp * 2 + jj
                    for m in range(GROUPS):
                        ids = featp_v[4 * jj, pl.ds(m * LANES, LANES)].astype(jnp.int32)
                        idx_v[0, pl.ds(m * LANES, LANES)] = lax.shift_right_logical(ids, 2)
                    pltpu.async_copy(
                        table_hbm.at[idx_v.at[0]], grows_v, sem
                    ).wait()
                    def asm(m, carry3):
                        chars = lax.iota(jnp.int32, LANES) + m * LANES
                        ids = featp_v[4 * jj, pl.ds(m * LANES, LANES)].astype(jnp.int32)
                        subs = lax.shift_left(lax.bitwise_and(ids, 3), 5)
                        for d in range(EMB_DIM):
                            vals = plsc.load_gather(grows_v, [chars, subs + d])
                            out_v[j * OUT_DIM + d, pl.ds(m * LANES, LANES)] = vals
                        for t in range(1, FEAT):
                            out_v[j * OUT_DIM + EMB_DIM + t - 1, pl.ds(m * LANES, LANES)] = (
                                featp_v[4 * jj + t, pl.ds(m * LANES, LANES)]
                            )
                        return carry3

                    lax.fori_loop(0, GROUPS, asm, 0)
                return carry2

            lax.fori_loop(0, WORDS // 2, jp_body, 0)
            pltpu.sync_copy(out_v, out_hbm.at[s, :, pl.ds(b0, B_BLOCK)])

        return carry

    lax.fori_loop(0, CHUNKS_PER_WORKER, chunk_body, 0)


def kernel(batch_features, embedding):
    batch_size, max_seq_length, flat_features = batch_features.shape
    max_word_length = flat_features // FEAT

    feat3 = jnp.transpose(batch_features, (1, 2, 0))
    table = embedding.reshape(embedding.shape[0] * EMB_DIM // 128, 128)

    call = pl.kernel(
        _sc_body,
        out_type=jax.ShapeDtypeStruct(
            (max_seq_length, max_word_length * OUT_DIM, batch_size), jnp.float32
        ),
        mesh=plsc.VectorSubcoreMesh(core_axis_name="c", subcore_axis_name="s"),
        scratch_types=[
            pltpu.VMEM((8, B_BLOCK), jnp.float32),
            pltpu.VMEM((1, B_BLOCK), jnp.int32),
            pltpu.VMEM((B_BLOCK, 128), jnp.float32),
            pltpu.VMEM((max_word_length * OUT_DIM, B_BLOCK), jnp.float32),
            pltpu.SemaphoreType.DMA,
        ],
        compiler_params=pltpu.CompilerParams(
            needs_layout_passes=False,
        ),
    )
    out3 = call(feat3, table)
    return jnp.transpose(out3, (2, 0, 1))
